# baseline (speedup 1.0000x reference)
import functools

import numpy as np

import jax
import jax.numpy as jnp
from jax.experimental import pallas as pl
from jax.experimental.pallas import tpu as pltpu
from jax.sharding import Mesh, PartitionSpec as P

_B, _S, _H = 2, 2048, 1024
_INTER = 2816
_TOPK = 8
_EPS = 1e-5
_LAYERS = 2

_BLK = 256


def _rms(y):
    var = jnp.mean(y * y, axis=-1, keepdims=True)
    return y * jax.lax.rsqrt(var + _EPS)


def _sparsify_body(g_ref, a_ref):
    g = g_ref[0]
    work = g
    col = jax.lax.broadcasted_iota(jnp.int32, g.shape, 1).astype(jnp.float32)
    big = float(_S)
    for _ in range(_TOPK):
        m = jnp.max(work, axis=-1, keepdims=True)
        cand = jnp.where(work == m, col, big)
        first = jnp.min(cand, axis=-1, keepdims=True)
        sel = cand == first
        work = jnp.where(sel, -jnp.inf, work)
    a_ref[0] = jnp.where(work == -jnp.inf, g, 0.0).astype(a_ref.dtype)


def _sparsify(graph):
    b = graph.shape[0]
    return pl.pallas_call(
        _sparsify_body,
        grid=(b, _S // _BLK),
        in_specs=[pl.BlockSpec((1, _BLK, _S), lambda b, i: (b, i, 0))],
        out_specs=pl.BlockSpec((1, _BLK, _S), lambda b, i: (b, i, 0)),
        out_shape=jax.ShapeDtypeStruct((b, _S, _S), jnp.bfloat16),
    )(graph)


def _inj_body(t_ref, wi_ref, o_ref):
    o_ref[...] = jnp.dot(t_ref[...], wi_ref[...],
                         preferred_element_type=jnp.float32)


def _inj_matmul(t2d, wi):
    n = t2d.shape[0]
    return pl.pallas_call(
        _inj_body,
        grid=(n // _BLK,),
        in_specs=[
            pl.BlockSpec((_BLK, _H), lambda i: (i, 0)),
            pl.BlockSpec((_H, _H), lambda i: (0, 0)),
        ],
        out_specs=pl.BlockSpec((_BLK, _H), lambda i: (i, 0)),
        out_shape=jax.ShapeDtypeStruct((n, _H), jnp.float32),
    )(t2d, wi)


def _msg_body(a_ref, xb_ref, x_ref, inj_ref, wm_ref, o_ref):
    t = jnp.dot(a_ref[0], xb_ref[0], preferred_element_type=jnp.float32)
    msg = jnp.dot(t.astype(jnp.bfloat16), wm_ref[...],
                  preferred_element_type=jnp.float32)
    y = x_ref[0] + inj_ref[0] + msg
    o_ref[0] = _rms(y)


def _msg_norm(a, x, inj, wm):
    b = x.shape[0]
    xb = x.astype(jnp.bfloat16)
    return pl.pallas_call(
        _msg_body,
        grid=(b, _S // _BLK),
        in_specs=[
            pl.BlockSpec((1, _BLK, _S), lambda b, i: (b, i, 0)),
            pl.BlockSpec((1, _S, _H), lambda b, i: (b, 0, 0)),
            pl.BlockSpec((1, _BLK, _H), lambda b, i: (b, i, 0)),
            pl.BlockSpec((1, _BLK, _H), lambda b, i: (b, i, 0)),
            pl.BlockSpec((_H, _H), lambda b, i: (0, 0)),
        ],
        out_specs=pl.BlockSpec((1, _BLK, _H), lambda b, i: (b, i, 0)),
        out_shape=jax.ShapeDtypeStruct((b, _S, _H), jnp.float32),
    )(a, xb, x, inj, wm)


def _swiglu_body(x_ref, wgu_ref, wd_ref, o_ref):
    x = x_ref[...]
    xb = x.astype(jnp.bfloat16)
    gu = jnp.dot(xb, wgu_ref[...], preferred_element_type=jnp.float32)
    gate, up = gu[:, :_INTER], gu[:, _INTER:]
    h = (jax.nn.silu(gate) * up).astype(jnp.bfloat16)
    mlp = jnp.dot(h, wd_ref[...], preferred_element_type=jnp.float32)
    y = x + mlp
    o_ref[...] = _rms(y)


def _swiglu(x2d, wgu, wd):
    n = x2d.shape[0]
    return pl.pallas_call(
        _swiglu_body,
        grid=(n // _BLK,),
        in_specs=[
            pl.BlockSpec((_BLK, _H), lambda i: (i, 0)),
            pl.BlockSpec((_H, 2 * _INTER), lambda i: (0, 0)),
            pl.BlockSpec((_INTER, _H), lambda i: (0, 0)),
        ],
        out_specs=pl.BlockSpec((_BLK, _H), lambda i: (i, 0)),
        out_shape=jax.ShapeDtypeStruct((n, _H), jnp.float32),
    )(x2d, wgu, wd)


def _forward(hidden_states, input_injection, graph, wi, wm, wgu, wd):
    b = hidden_states.shape[0]
    a = _sparsify(graph)
    inj = _inj_matmul(
        input_injection.reshape(b * _S, _H).astype(jnp.bfloat16), wi
    ).reshape(b, _S, _H)
    x = hidden_states
    for _ in range(_LAYERS):
        x = _msg_norm(a, x, inj, wm)
        x = _swiglu(x.reshape(b * _S, _H), wgu, wd).reshape(b, _S, _H)
    return x


@jax.jit
def kernel(hidden_states, input_injection, graph, Wi, Wm, Wgu, Wd):
    wi = Wi.astype(jnp.bfloat16)
    wm = Wm.astype(jnp.bfloat16)
    wgu = Wgu.astype(jnp.bfloat16)
    wd = Wd.astype(jnp.bfloat16)
    n_dev = min(_B, len(jax.devices()))
    mesh = Mesh(np.array(jax.devices()[:n_dev]), ("b",))
    f = jax.shard_map(
        _forward,
        mesh=mesh,
        in_specs=(P("b"), P("b"), P("b"), P(), P(), P(), P()),
        out_specs=P("b"),
        check_vma=False,
    )
    return f(hidden_states, input_injection, graph, wi, wm, wgu, wd)


# speedup vs baseline: 1.9640x; 1.9640x over previous
import functools

import numpy as np

import jax
import jax.numpy as jnp
from jax.experimental import pallas as pl
from jax.experimental.pallas import tpu as pltpu
from jax.sharding import Mesh, PartitionSpec as P

_B, _S, _H = 2, 2048, 1024
_INTER = 2816
_TOPK = 8
_EPS = 1e-5
_LAYERS = 2

_BLK = 256


def _rms(y):
    var = jnp.mean(y * y, axis=-1, keepdims=True)
    return y * jax.lax.rsqrt(var + _EPS)


def _sparsify_body(g_ref, a_ref):
    g = g_ref[0]
    work = g
    col = jax.lax.broadcasted_iota(jnp.int32, g.shape, 1).astype(jnp.float32)
    big = float(_S)
    for _ in range(_TOPK):
        m = jnp.max(work, axis=-1, keepdims=True)
        cand = jnp.where(work == m, col, big)
        first = jnp.min(cand, axis=-1, keepdims=True)
        sel = cand == first
        work = jnp.where(sel, -jnp.inf, work)
    a_ref[0] = jnp.where(work == -jnp.inf, g, 0.0).astype(a_ref.dtype)


def _sparsify(graph):
    b = graph.shape[0]
    return pl.pallas_call(
        _sparsify_body,
        grid=(b, _S // _BLK),
        in_specs=[pl.BlockSpec((1, _BLK, _S), lambda b, i: (b, i, 0))],
        out_specs=pl.BlockSpec((1, _BLK, _S), lambda b, i: (b, i, 0)),
        out_shape=jax.ShapeDtypeStruct((b, _S, _S), jnp.bfloat16),
    )(graph)


def _inj_body(t_ref, wi_ref, o_ref):
    o_ref[...] = jnp.dot(t_ref[...], wi_ref[...],
                         preferred_element_type=jnp.float32)


def _inj_matmul(t2d, wi):
    n = t2d.shape[0]
    return pl.pallas_call(
        _inj_body,
        grid=(n // _BLK,),
        in_specs=[
            pl.BlockSpec((_BLK, _H), lambda i: (i, 0)),
            pl.BlockSpec((_H, _H), lambda i: (0, 0)),
        ],
        out_specs=pl.BlockSpec((_BLK, _H), lambda i: (i, 0)),
        out_shape=jax.ShapeDtypeStruct((n, _H), jnp.float32),
    )(t2d, wi)


def _msg_body(a_ref, xb_ref, x_ref, inj_ref, wm_ref, o_ref):
    t = jnp.dot(a_ref[0], xb_ref[0], preferred_element_type=jnp.float32)
    msg = jnp.dot(t.astype(jnp.bfloat16), wm_ref[...],
                  preferred_element_type=jnp.float32)
    y = x_ref[0] + inj_ref[0] + msg
    o_ref[0] = _rms(y)


def _msg_norm(a, x, inj, wm):
    b = x.shape[0]
    xb = x.astype(jnp.bfloat16)
    return pl.pallas_call(
        _msg_body,
        grid=(b, _S // _BLK),
        in_specs=[
            pl.BlockSpec((1, _BLK, _S), lambda b, i: (b, i, 0)),
            pl.BlockSpec((1, _S, _H), lambda b, i: (b, 0, 0)),
            pl.BlockSpec((1, _BLK, _H), lambda b, i: (b, i, 0)),
            pl.BlockSpec((1, _BLK, _H), lambda b, i: (b, i, 0)),
            pl.BlockSpec((_H, _H), lambda b, i: (0, 0)),
        ],
        out_specs=pl.BlockSpec((1, _BLK, _H), lambda b, i: (b, i, 0)),
        out_shape=jax.ShapeDtypeStruct((b, _S, _H), jnp.float32),
    )(a, xb, x, inj, wm)


def _swiglu_body(x_ref, wgu_ref, wd_ref, o_ref):
    x = x_ref[...]
    xb = x.astype(jnp.bfloat16)
    gu = jnp.dot(xb, wgu_ref[...], preferred_element_type=jnp.float32)
    gate, up = gu[:, :_INTER], gu[:, _INTER:]
    h = (jax.nn.silu(gate) * up).astype(jnp.bfloat16)
    mlp = jnp.dot(h, wd_ref[...], preferred_element_type=jnp.float32)
    y = x + mlp
    o_ref[...] = _rms(y)


def _swiglu(x2d, wgu, wd):
    n = x2d.shape[0]
    return pl.pallas_call(
        _swiglu_body,
        grid=(n // _BLK,),
        in_specs=[
            pl.BlockSpec((_BLK, _H), lambda i: (i, 0)),
            pl.BlockSpec((_H, 2 * _INTER), lambda i: (0, 0)),
            pl.BlockSpec((_INTER, _H), lambda i: (0, 0)),
        ],
        out_specs=pl.BlockSpec((_BLK, _H), lambda i: (i, 0)),
        out_shape=jax.ShapeDtypeStruct((n, _H), jnp.float32),
    )(x2d, wgu, wd)


def _forward(hidden_states, input_injection, graph, wi, wm, wgu, wd):
    b = hidden_states.shape[0]
    a = _sparsify(graph)
    inj = _inj_matmul(
        input_injection.reshape(b * _S, _H).astype(jnp.bfloat16), wi
    ).reshape(b, _S, _H)
    x = hidden_states
    for _ in range(_LAYERS):
        x = _msg_norm(a, x, inj, wm)
        x = _swiglu(x.reshape(b * _S, _H), wgu, wd).reshape(b, _S, _H)
    return x


@jax.jit
def kernel(hidden_states, input_injection, graph, Wi, Wm, Wgu, Wd):
    wi = Wi.astype(jnp.bfloat16)
    wm = Wm.astype(jnp.bfloat16)
    wgu = Wgu.astype(jnp.bfloat16)
    wd = Wd.astype(jnp.bfloat16)
    return _forward(hidden_states, input_injection, graph, wi, wm, wgu, wd)


# speedup vs baseline: 2.0045x; 1.0206x over previous
import jax
import jax.numpy as jnp
from jax.experimental import pallas as pl
from jax.experimental.pallas import tpu as pltpu

_B, _S, _H = 2, 2048, 1024
_INTER = 2816
_TOPK = 8
_EPS = 1e-5
_LAYERS = 2

_BLK = 256
_N = _B * _S // _BLK
_PB = _S // _BLK


def _rms(y):
    var = jnp.mean(y * y, axis=-1, keepdims=True)
    return y * jax.lax.rsqrt(var + _EPS)


def _sparsify_body(g_ref, a_ref):
    g = g_ref[0]
    work = g
    col = jax.lax.broadcasted_iota(jnp.int32, g.shape, 1).astype(jnp.float32)
    big = float(_S)
    for _ in range(_TOPK):
        m = jnp.max(work, axis=-1, keepdims=True)
        cand = jnp.where(work == m, col, big)
        first = jnp.min(cand, axis=-1, keepdims=True)
        sel = cand == first
        work = jnp.where(sel, -jnp.inf, work)
    a_ref[0] = jnp.where(work == -jnp.inf, g, 0.0).astype(a_ref.dtype)


def _sparsify(graph):
    return pl.pallas_call(
        _sparsify_body,
        grid=(_B, _PB),
        in_specs=[pl.BlockSpec((1, _BLK, _S), lambda b, i: (b, i, 0))],
        out_specs=pl.BlockSpec((1, _BLK, _S), lambda b, i: (b, i, 0)),
        out_shape=jax.ShapeDtypeStruct((_B, _S, _S), jnp.bfloat16),
    )(graph)


def _inj_body(t_ref, wi_ref, o_ref):
    o_ref[...] = jnp.dot(t_ref[...], wi_ref[...],
                         preferred_element_type=jnp.float32)


def _inj_matmul(t2d, wi):
    n = t2d.shape[0]
    return pl.pallas_call(
        _inj_body,
        grid=(n // _BLK,),
        in_specs=[
            pl.BlockSpec((_BLK, _H), lambda i: (i, 0)),
            pl.BlockSpec((_H, _H), lambda i: (0, 0)),
        ],
        out_specs=pl.BlockSpec((_BLK, _H), lambda i: (i, 0)),
        out_shape=jax.ShapeDtypeStruct((n, _H), jnp.float32),
    )(t2d, wi)


def _layer_body(a_ref, xb_ref, x_ref, inj_ref, wm_ref, wgu_ref, wd_ref,
                o_ref, ob_ref, xn_ref):
    s = pl.program_id(0)

    t = jnp.dot(a_ref[0], xb_ref[0], preferred_element_type=jnp.float32)
    msg = jnp.dot(t.astype(jnp.bfloat16), wm_ref[...],
                  preferred_element_type=jnp.float32)
    xn = _rms(x_ref[0] + inj_ref[0] + msg)
    xn_ref[s % 2] = xn

    x1 = xn_ref[(s + 1) % 2]
    x1b = x1.astype(jnp.bfloat16)
    gu = jnp.dot(x1b, wgu_ref[...], preferred_element_type=jnp.float32)
    gate, up = gu[:, :_INTER], gu[:, _INTER:]
    h = (jax.nn.silu(gate) * up).astype(jnp.bfloat16)
    mlp = jnp.dot(h, wd_ref[...], preferred_element_type=jnp.float32)
    y = _rms(x1 + mlp)
    o_ref[0] = y
    ob_ref[0] = y.astype(jnp.bfloat16)


def _layer(a, xb, x, inj, wm, wgu, wd):
    def in_idx(s):
        c = jnp.minimum(s, _N - 1)
        return (c // _PB, c % _PB, 0)

    def slab_idx(s):
        c = jnp.minimum(s, _N - 1)
        return (c // _PB, 0, 0)

    def out_idx(s):
        c = jnp.maximum(s - 1, 0)
        return (c // _PB, c % _PB, 0)

    return pl.pallas_call(
        _layer_body,
        grid=(_N + 1,),
        in_specs=[
            pl.BlockSpec((1, _BLK, _S), in_idx),
            pl.BlockSpec((1, _S, _H), slab_idx),
            pl.BlockSpec((1, _BLK, _H), in_idx),
            pl.BlockSpec((1, _BLK, _H), in_idx),
            pl.BlockSpec((_H, _H), lambda s: (0, 0)),
            pl.BlockSpec((_H, 2 * _INTER), lambda s: (0, 0)),
            pl.BlockSpec((_INTER, _H), lambda s: (0, 0)),
        ],
        out_specs=[
            pl.BlockSpec((1, _BLK, _H), out_idx),
            pl.BlockSpec((1, _BLK, _H), out_idx),
        ],
        out_shape=[
            jax.ShapeDtypeStruct((_B, _S, _H), jnp.float32),
            jax.ShapeDtypeStruct((_B, _S, _H), jnp.bfloat16),
        ],
        scratch_shapes=[pltpu.VMEM((2, _BLK, _H), jnp.float32)],
    )(a, xb, x, inj, wm, wgu, wd)


@jax.jit
def kernel(hidden_states, input_injection, graph, Wi, Wm, Wgu, Wd):
    wi = Wi.astype(jnp.bfloat16)
    wm = Wm.astype(jnp.bfloat16)
    wgu = Wgu.astype(jnp.bfloat16)
    wd = Wd.astype(jnp.bfloat16)
    a = _sparsify(graph)
    inj = _inj_matmul(
        input_injection.reshape(_B * _S, _H).astype(jnp.bfloat16), wi
    ).reshape(_B, _S, _H)
    x = hidden_states
    xb = x.astype(jnp.bfloat16)
    for _ in range(_LAYERS):
        x, xb = _layer(a, xb, x, inj, wm, wgu, wd)
    return x


# speedup vs baseline: 2.0333x; 1.0143x over previous
import jax
import jax.numpy as jnp
from jax.experimental import pallas as pl
from jax.experimental.pallas import tpu as pltpu

_B, _S, _H = 2, 2048, 1024
_INTER = 2816
_TOPK = 8
_EPS = 1e-5
_LAYERS = 2

_BLK = 256
_SB = 512
_NS = _B * _S // _SB
_PSB = _S // _SB


def _rms(y):
    var = jnp.mean(y * y, axis=-1, keepdims=True)
    return y * jax.lax.rsqrt(var + _EPS)


def _sparsify_body(g_ref, a_ref):
    g = g_ref[0]
    work = g
    col = jax.lax.broadcasted_iota(jnp.int32, g.shape, 1).astype(jnp.float32)
    big = float(_S)
    for _ in range(_TOPK):
        m = jnp.max(work, axis=-1, keepdims=True)
        cand = jnp.where(work == m, col, big)
        first = jnp.min(cand, axis=-1, keepdims=True)
        sel = cand == first
        work = jnp.where(sel, -jnp.inf, work)
    a_ref[0] = jnp.where(work == -jnp.inf, g, 0.0).astype(a_ref.dtype)


def _sparsify(graph):
    return pl.pallas_call(
        _sparsify_body,
        grid=(_B, _S // _BLK),
        in_specs=[pl.BlockSpec((1, _BLK, _S), lambda b, i: (b, i, 0))],
        out_specs=pl.BlockSpec((1, _BLK, _S), lambda b, i: (b, i, 0)),
        out_shape=jax.ShapeDtypeStruct((_B, _S, _S), jnp.bfloat16),
    )(graph)


def _inj_body(t_ref, wi_ref, o_ref):
    o_ref[...] = jnp.dot(t_ref[...], wi_ref[...],
                         preferred_element_type=jnp.float32)


def _inj_matmul(t2d, wi):
    n = t2d.shape[0]
    return pl.pallas_call(
        _inj_body,
        grid=(n // _BLK,),
        in_specs=[
            pl.BlockSpec((_BLK, _H), lambda i: (i, 0)),
            pl.BlockSpec((_H, _H), lambda i: (0, 0)),
        ],
        out_specs=pl.BlockSpec((_BLK, _H), lambda i: (i, 0)),
        out_shape=jax.ShapeDtypeStruct((n, _H), jnp.float32),
    )(t2d, wi)


def _swiglu_block(x1, wgu_ref, wd_ref):
    x1b = x1.astype(jnp.bfloat16)
    gu = jnp.dot(x1b, wgu_ref[...], preferred_element_type=jnp.float32)
    gate, up = gu[:, :_INTER], gu[:, _INTER:]
    h = (jax.nn.silu(gate) * up).astype(jnp.bfloat16)
    mlp = jnp.dot(h, wd_ref[...], preferred_element_type=jnp.float32)
    return _rms(x1 + mlp)


def _msg_block(a, xb_ref, x, inj, wm_ref):
    t = jnp.dot(a, xb_ref[0], preferred_element_type=jnp.float32)
    msg = jnp.dot(t.astype(jnp.bfloat16), wm_ref[...],
                  preferred_element_type=jnp.float32)
    return _rms(x.astype(jnp.float32) + inj + msg)


def _layer_body(a_ref, xb_ref, inj_ref, wm_ref, wgu_ref, wd_ref,
                o_ref, ob_ref, sc_ref, sd_ref):
    s = pl.program_id(0)
    c = jnp.minimum(s, _NS - 1)
    off = (c % _PSB) * _SB

    x1_0 = sc_ref[...]
    x1_1 = sd_ref[...]
    y0 = _swiglu_block(x1_0, wgu_ref, wd_ref)
    y1 = _swiglu_block(x1_1, wgu_ref, wd_ref)
    o_ref[0, :_BLK] = y0
    o_ref[0, _BLK:] = y1
    ob_ref[0, :_BLK] = y0.astype(jnp.bfloat16)
    ob_ref[0, _BLK:] = y1.astype(jnp.bfloat16)

    sc_ref[...] = _msg_block(a_ref[0, :_BLK], xb_ref,
                             xb_ref[0, pl.ds(off, _BLK)],
                             inj_ref[0, :_BLK], wm_ref)
    sd_ref[...] = _msg_block(a_ref[0, _BLK:], xb_ref,
                             xb_ref[0, pl.ds(off + _BLK, _BLK)],
                             inj_ref[0, _BLK:], wm_ref)


def _layer(a, xb, inj, wm, wgu, wd):
    def in_idx(s):
        c = jnp.minimum(s, _NS - 1)
        return (c // _PSB, c % _PSB, 0)

    def slab_idx(s):
        c = jnp.minimum(s, _NS - 1)
        return (c // _PSB, 0, 0)

    def out_idx(s):
        c = jnp.maximum(s - 1, 0)
        return (c // _PSB, c % _PSB, 0)

    return pl.pallas_call(
        _layer_body,
        grid=(_NS + 1,),
        in_specs=[
            pl.BlockSpec((1, _SB, _S), in_idx),
            pl.BlockSpec((1, _S, _H), slab_idx),
            pl.BlockSpec((1, _SB, _H), in_idx),
            pl.BlockSpec((_H, _H), lambda s: (0, 0)),
            pl.BlockSpec((_H, 2 * _INTER), lambda s: (0, 0)),
            pl.BlockSpec((_INTER, _H), lambda s: (0, 0)),
        ],
        out_specs=[
            pl.BlockSpec((1, _SB, _H), out_idx),
            pl.BlockSpec((1, _SB, _H), out_idx),
        ],
        out_shape=[
            jax.ShapeDtypeStruct((_B, _S, _H), jnp.float32),
            jax.ShapeDtypeStruct((_B, _S, _H), jnp.bfloat16),
        ],
        scratch_shapes=[
            pltpu.VMEM((_BLK, _H), jnp.float32),
            pltpu.VMEM((_BLK, _H), jnp.float32),
        ],
    )(a, xb, inj, wm, wgu, wd)


@jax.jit
def kernel(hidden_states, input_injection, graph, Wi, Wm, Wgu, Wd):
    wi = Wi.astype(jnp.bfloat16)
    wm = Wm.astype(jnp.bfloat16)
    wgu = Wgu.astype(jnp.bfloat16)
    wd = Wd.astype(jnp.bfloat16)
    a = _sparsify(graph)
    inj = _inj_matmul(
        input_injection.reshape(_B * _S, _H).astype(jnp.bfloat16), wi
    ).reshape(_B, _S, _H)
    xb = hidden_states.astype(jnp.bfloat16)
    x = None
    for _ in range(_LAYERS):
        x, xb = _layer(a, xb, inj, wm, wgu, wd)
    return x
